# baseline (speedup 1.0000x reference)
import functools

import jax
import jax.numpy as jnp
from jax.experimental import pallas as pl
from jax.experimental.pallas import tpu as pltpu

NUM_EXPERTS = 64
HIDDEN = 768
EXPERT_CAPACITY = 128
BLOCK_S = 2048


def _router_kernel(x_ref, wt_ref, tril_ref, dispatch_ref, combine_ref,
                   probs_ref, aux_ref, z_ref, carry_ref, acc_ref, *,
                   nsb, total_tokens):
    b = pl.program_id(0)
    sblk = pl.program_id(1)
    T = x_ref.shape[1]
    E = NUM_EXPERTS

    x = x_ref[0]
    logits = jax.lax.dot_general(
        x, wt_ref[...], (((1,), (0,)), ((), ())),
        preferred_element_type=jnp.float32)

    m = jnp.max(logits, axis=-1, keepdims=True)
    ex = jnp.exp(logits - m)
    denom = jnp.sum(ex, axis=-1, keepdims=True)
    probs = ex / denom
    probs_ref[0] = probs

    maxp = jnp.max(probs, axis=-1, keepdims=True)
    lane = jax.lax.broadcasted_iota(jnp.int32, (T, E), 1)
    idx = jnp.min(jnp.where(probs == maxp, lane, E), axis=-1, keepdims=True)
    mask = (lane == idx).astype(jnp.float32)

    @pl.when(sblk == 0)
    def _():
        carry_ref[...] = jnp.zeros_like(carry_ref)

    csum = jax.lax.dot_general(
        tril_ref[...], mask, (((1,), (0,)), ((), ())),
        preferred_element_type=jnp.float32) + carry_ref[...]
    carry_ref[...] = csum[T - 1:T, :]

    dispatch = jnp.where((mask > 0) & (csum <= EXPERT_CAPACITY), 1.0, 0.0)
    dispatch_ref[0] = dispatch
    combine_ref[0] = dispatch * maxp

    @pl.when((b == 0) & (sblk == 0))
    def _():
        acc_ref[0] = 0.0
        acc_ref[1] = 0.0

    lse = m + jnp.log(denom)
    acc_ref[0] += jnp.sum(probs * probs)
    acc_ref[1] += jnp.sum(lse * lse)
    aux_ref[...] = jnp.full((1, 1), acc_ref[0] * (E / total_tokens),
                            jnp.float32)
    z_ref[...] = jnp.full((1, 1), acc_ref[1] / total_tokens, jnp.float32)


@jax.jit
def kernel(hidden_states, W):
    B, S, D = hidden_states.shape
    E = W.shape[0]
    nsb = S // BLOCK_S
    wt = W.T
    r = jax.lax.broadcasted_iota(jnp.int32, (BLOCK_S, BLOCK_S), 0)
    c = jax.lax.broadcasted_iota(jnp.int32, (BLOCK_S, BLOCK_S), 1)
    tril = (r >= c).astype(jnp.float32)

    out_shapes = (
        jax.ShapeDtypeStruct((B, S, E), jnp.float32),
        jax.ShapeDtypeStruct((B, S, E), jnp.float32),
        jax.ShapeDtypeStruct((B, S, E), jnp.float32),
        jax.ShapeDtypeStruct((1, 1), jnp.float32),
        jax.ShapeDtypeStruct((1, 1), jnp.float32),
    )
    bse_spec = pl.BlockSpec((1, BLOCK_S, E), lambda b, s: (b, s, 0))
    scalar_spec = pl.BlockSpec((1, 1), lambda b, s: (0, 0))

    dispatch, combine, probs, aux, z = pl.pallas_call(
        functools.partial(_router_kernel, nsb=nsb, total_tokens=B * S),
        grid=(B, nsb),
        in_specs=[
            pl.BlockSpec((1, BLOCK_S, D), lambda b, s: (b, s, 0)),
            pl.BlockSpec((D, E), lambda b, s: (0, 0)),
            pl.BlockSpec((BLOCK_S, BLOCK_S), lambda b, s: (0, 0)),
        ],
        out_specs=(bse_spec, bse_spec, bse_spec, scalar_spec, scalar_spec),
        out_shape=out_shapes,
        scratch_shapes=[
            pltpu.VMEM((1, E), jnp.float32),
            pltpu.SMEM((2,), jnp.float32),
        ],
    )(hidden_states, wt, tril)

    return (dispatch, combine, probs, aux[0, 0], z[0, 0])


# speedup vs baseline: 1.4178x; 1.4178x over previous
import functools

import jax
import jax.numpy as jnp
from jax.experimental import pallas as pl
from jax.experimental.pallas import tpu as pltpu

NUM_EXPERTS = 64
HIDDEN = 768
EXPERT_CAPACITY = 128
BLOCK_S = 1024
CHUNK = 128


def _router_kernel(x_ref, wt_ref, tril_ref, dispatch_ref, combine_ref,
                   probs_ref, aux_ref, z_ref, carry_ref, acc_ref, *,
                   total_tokens):
    b = pl.program_id(0)
    sblk = pl.program_id(1)
    T = x_ref.shape[1]
    E = NUM_EXPERTS

    x = x_ref[0]
    logits = jax.lax.dot_general(
        x, wt_ref[...], (((1,), (0,)), ((), ())),
        preferred_element_type=jnp.float32)

    m = jnp.max(logits, axis=-1, keepdims=True)
    ex = jnp.exp(logits - m)
    denom = jnp.sum(ex, axis=-1, keepdims=True)
    probs = ex / denom
    probs_ref[0] = probs

    maxp = 1.0 / denom
    lane = jax.lax.broadcasted_iota(jnp.int32, (T, E), 1)
    idx = jnp.min(jnp.where(probs == maxp, lane, E), axis=-1, keepdims=True)
    onehot = lane == idx
    mask = onehot.astype(jnp.float32)

    @pl.when(sblk == 0)
    def _():
        carry_ref[...] = jnp.zeros_like(carry_ref)

    run = carry_ref[...]
    for c in range(T // CHUNK):
        sl = slice(c * CHUNK, (c + 1) * CHUNK)
        csum_c = jax.lax.dot_general(
            tril_ref[...], mask[sl], (((1,), (0,)), ((), ())),
            preferred_element_type=jnp.float32) + run
        d_c = jnp.where(onehot[sl] & (csum_c <= EXPERT_CAPACITY), 1.0, 0.0)
        dispatch_ref[0, sl, :] = d_c
        combine_ref[0, sl, :] = d_c * maxp[sl]
        run = csum_c[CHUNK - 1:CHUNK, :]
    carry_ref[...] = run

    @pl.when((b == 0) & (sblk == 0))
    def _():
        acc_ref[0] = 0.0
        acc_ref[1] = 0.0

    lse = m + jnp.log(denom)
    acc_ref[0] += jnp.sum(probs * probs)
    acc_ref[1] += jnp.sum(lse * lse)
    aux_ref[...] = jnp.full((1, 1), acc_ref[0] * (E / total_tokens),
                            jnp.float32)
    z_ref[...] = jnp.full((1, 1), acc_ref[1] / total_tokens, jnp.float32)


@jax.jit
def kernel(hidden_states, W):
    B, S, D = hidden_states.shape
    E = W.shape[0]
    nsb = S // BLOCK_S
    wt = W.T
    r = jax.lax.broadcasted_iota(jnp.int32, (CHUNK, CHUNK), 0)
    c = jax.lax.broadcasted_iota(jnp.int32, (CHUNK, CHUNK), 1)
    tril = (r >= c).astype(jnp.float32)

    out_shapes = (
        jax.ShapeDtypeStruct((B, S, E), jnp.float32),
        jax.ShapeDtypeStruct((B, S, E), jnp.float32),
        jax.ShapeDtypeStruct((B, S, E), jnp.float32),
        jax.ShapeDtypeStruct((1, 1), jnp.float32),
        jax.ShapeDtypeStruct((1, 1), jnp.float32),
    )
    bse_spec = pl.BlockSpec((1, BLOCK_S, E), lambda b, s: (b, s, 0))
    scalar_spec = pl.BlockSpec((1, 1), lambda b, s: (0, 0))

    dispatch, combine, probs, aux, z = pl.pallas_call(
        functools.partial(_router_kernel, total_tokens=B * S),
        grid=(B, nsb),
        in_specs=[
            pl.BlockSpec((1, BLOCK_S, D), lambda b, s: (b, s, 0)),
            pl.BlockSpec((D, E), lambda b, s: (0, 0)),
            pl.BlockSpec((CHUNK, CHUNK), lambda b, s: (0, 0)),
        ],
        out_specs=(bse_spec, bse_spec, bse_spec, scalar_spec, scalar_spec),
        out_shape=out_shapes,
        scratch_shapes=[
            pltpu.VMEM((1, E), jnp.float32),
            pltpu.SMEM((2,), jnp.float32),
        ],
    )(hidden_states, wt, tril)

    return (dispatch, combine, probs, aux[0, 0], z[0, 0])


# speedup vs baseline: 1.4665x; 1.0344x over previous
import functools

import jax
import jax.numpy as jnp
from jax.experimental import pallas as pl
from jax.experimental.pallas import tpu as pltpu

NUM_EXPERTS = 64
HIDDEN = 768
EXPERT_CAPACITY = 128
BLOCK_S = 1024
CHUNK = 128


def _router_kernel(x_ref, wt_ref, tril_ref, dispatch_ref, combine_ref,
                   probs_ref, aux_ref, z_ref, carry_ref, acc_ref, *,
                   total_tokens):
    b = pl.program_id(0)
    sblk = pl.program_id(1)
    T = x_ref.shape[1]
    E = NUM_EXPERTS

    x = x_ref[0]
    logits = jax.lax.dot_general(
        x, wt_ref[...], (((1,), (0,)), ((), ())),
        preferred_element_type=jnp.float32)

    m = jnp.max(logits, axis=-1, keepdims=True)
    ex = jnp.exp(logits - m)
    denom = jnp.sum(ex, axis=-1, keepdims=True)
    probs = ex / denom
    probs_ref[0] = probs

    maxp = 1.0 / denom
    lane = jax.lax.broadcasted_iota(jnp.int32, (T, E), 1).astype(jnp.float32)
    idx = jnp.min(jnp.where(probs == maxp, lane, float(E)),
                  axis=-1, keepdims=True)
    mask = (lane == idx).astype(jnp.float32)

    @pl.when(sblk == 0)
    def _():
        carry_ref[...] = jnp.zeros_like(carry_ref)

    run = carry_ref[...]
    for c in range(T // CHUNK):
        sl = slice(c * CHUNK, (c + 1) * CHUNK)
        csum_c = jax.lax.dot_general(
            tril_ref[...], mask[sl], (((1,), (0,)), ((), ())),
            preferred_element_type=jnp.float32) + run
        d_c = jnp.where(csum_c <= EXPERT_CAPACITY, mask[sl], 0.0)
        dispatch_ref[0, sl, :] = d_c
        combine_ref[0, sl, :] = d_c * maxp[sl]
        run = csum_c[CHUNK - 1:CHUNK, :]
    carry_ref[...] = run

    @pl.when((b == 0) & (sblk == 0))
    def _():
        acc_ref[0] = 0.0
        acc_ref[1] = 0.0

    lse = m + jnp.log(denom)
    acc_ref[0] += jnp.sum(probs * probs)
    acc_ref[1] += jnp.sum(lse * lse)
    aux_ref[...] = jnp.full((1, 1), acc_ref[0] * (E / total_tokens),
                            jnp.float32)
    z_ref[...] = jnp.full((1, 1), acc_ref[1] / total_tokens, jnp.float32)


@jax.jit
def kernel(hidden_states, W):
    B, S, D = hidden_states.shape
    E = W.shape[0]
    nsb = S // BLOCK_S
    wt = W.T
    r = jax.lax.broadcasted_iota(jnp.int32, (CHUNK, CHUNK), 0)
    c = jax.lax.broadcasted_iota(jnp.int32, (CHUNK, CHUNK), 1)
    tril = (r >= c).astype(jnp.float32)

    out_shapes = (
        jax.ShapeDtypeStruct((B, S, E), jnp.float32),
        jax.ShapeDtypeStruct((B, S, E), jnp.float32),
        jax.ShapeDtypeStruct((B, S, E), jnp.float32),
        jax.ShapeDtypeStruct((1, 1), jnp.float32),
        jax.ShapeDtypeStruct((1, 1), jnp.float32),
    )
    bse_spec = pl.BlockSpec((1, BLOCK_S, E), lambda b, s: (b, s, 0))
    scalar_spec = pl.BlockSpec((1, 1), lambda b, s: (0, 0))

    dispatch, combine, probs, aux, z = pl.pallas_call(
        functools.partial(_router_kernel, total_tokens=B * S),
        grid=(B, nsb),
        in_specs=[
            pl.BlockSpec((1, BLOCK_S, D), lambda b, s: (b, s, 0)),
            pl.BlockSpec((D, E), lambda b, s: (0, 0)),
            pl.BlockSpec((CHUNK, CHUNK), lambda b, s: (0, 0)),
        ],
        out_specs=(bse_spec, bse_spec, bse_spec, scalar_spec, scalar_spec),
        out_shape=out_shapes,
        scratch_shapes=[
            pltpu.VMEM((1, E), jnp.float32),
            pltpu.SMEM((2,), jnp.float32),
        ],
    )(hidden_states, wt, tril)

    return (dispatch, combine, probs, aux[0, 0], z[0, 0])


# speedup vs baseline: 1.6261x; 1.1088x over previous
import functools

import jax
import jax.numpy as jnp
from jax.experimental import pallas as pl
from jax.experimental.pallas import tpu as pltpu

NUM_EXPERTS = 64
HIDDEN = 768
EXPERT_CAPACITY = 128
BLOCK_S = 2048
CHUNK = 128


def _router_kernel(x_ref, wt_ref, tril_ref, dispatch_ref, combine_ref,
                   probs_ref, aux_ref, z_ref, carry_ref, acc_ref, *,
                   total_tokens):
    b = pl.program_id(0)
    sblk = pl.program_id(1)
    T = x_ref.shape[1]
    E = NUM_EXPERTS

    x = x_ref[0]
    logits = jax.lax.dot_general(
        x, wt_ref[...], (((1,), (0,)), ((), ())),
        preferred_element_type=jnp.float32)

    m = jnp.max(logits, axis=-1, keepdims=True)
    ex = jnp.exp(logits - m)
    denom = jnp.sum(ex, axis=-1, keepdims=True)
    probs = ex / denom
    probs_ref[0] = probs

    maxp = 1.0 / denom
    lane = jax.lax.broadcasted_iota(jnp.int32, (T, E), 1).astype(jnp.float32)
    idx = jnp.min(jnp.where(probs == maxp, lane, float(E)),
                  axis=-1, keepdims=True)
    mask = (lane == idx).astype(jnp.float32)

    @pl.when(sblk == 0)
    def _():
        carry_ref[...] = jnp.zeros_like(carry_ref)

    run = carry_ref[...]
    for c in range(T // CHUNK):
        sl = slice(c * CHUNK, (c + 1) * CHUNK)
        csum_c = jax.lax.dot_general(
            tril_ref[...], mask[sl], (((1,), (0,)), ((), ())),
            preferred_element_type=jnp.float32) + run
        d_c = jnp.where(csum_c <= EXPERT_CAPACITY, mask[sl], 0.0)
        dispatch_ref[0, sl, :] = d_c
        combine_ref[0, sl, :] = d_c * maxp[sl]
        run = csum_c[CHUNK - 1:CHUNK, :]
    carry_ref[...] = run

    @pl.when((b == 0) & (sblk == 0))
    def _():
        acc_ref[0] = 0.0
        acc_ref[1] = 0.0

    lse = m + jnp.log(denom)
    acc_ref[0] += jnp.sum(probs * probs)
    acc_ref[1] += jnp.sum(lse * lse)
    aux_ref[...] = jnp.full((1, 1), acc_ref[0] * (E / total_tokens),
                            jnp.float32)
    z_ref[...] = jnp.full((1, 1), acc_ref[1] / total_tokens, jnp.float32)


@jax.jit
def kernel(hidden_states, W):
    B, S, D = hidden_states.shape
    E = W.shape[0]
    nsb = S // BLOCK_S
    wt = W.T
    r = jax.lax.broadcasted_iota(jnp.int32, (CHUNK, CHUNK), 0)
    c = jax.lax.broadcasted_iota(jnp.int32, (CHUNK, CHUNK), 1)
    tril = (r >= c).astype(jnp.float32)

    out_shapes = (
        jax.ShapeDtypeStruct((B, S, E), jnp.float32),
        jax.ShapeDtypeStruct((B, S, E), jnp.float32),
        jax.ShapeDtypeStruct((B, S, E), jnp.float32),
        jax.ShapeDtypeStruct((1, 1), jnp.float32),
        jax.ShapeDtypeStruct((1, 1), jnp.float32),
    )
    bse_spec = pl.BlockSpec((1, BLOCK_S, E), lambda b, s: (b, s, 0))
    scalar_spec = pl.BlockSpec((1, 1), lambda b, s: (0, 0))

    dispatch, combine, probs, aux, z = pl.pallas_call(
        functools.partial(_router_kernel, total_tokens=B * S),
        grid=(B, nsb),
        in_specs=[
            pl.BlockSpec((1, BLOCK_S, D), lambda b, s: (b, s, 0)),
            pl.BlockSpec((D, E), lambda b, s: (0, 0)),
            pl.BlockSpec((CHUNK, CHUNK), lambda b, s: (0, 0)),
        ],
        out_specs=(bse_spec, bse_spec, bse_spec, scalar_spec, scalar_spec),
        out_shape=out_shapes,
        scratch_shapes=[
            pltpu.VMEM((1, E), jnp.float32),
            pltpu.SMEM((2,), jnp.float32),
        ],
    )(hidden_states, wt, tril)

    return (dispatch, combine, probs, aux[0, 0], z[0, 0])


# speedup vs baseline: 1.6691x; 1.0264x over previous
import functools

import jax
import jax.numpy as jnp
from jax.experimental import pallas as pl
from jax.experimental.pallas import tpu as pltpu

NUM_EXPERTS = 64
HIDDEN = 768
EXPERT_CAPACITY = 128
BLOCK_S = 4096
CHUNK = 128


def _router_kernel(x_ref, wt_ref, tril_ref, dispatch_ref, combine_ref,
                   probs_ref, aux_ref, z_ref, carry_ref, acc_ref, *,
                   total_tokens):
    b = pl.program_id(0)
    sblk = pl.program_id(1)
    T = x_ref.shape[1]
    E = NUM_EXPERTS

    x = x_ref[0]
    logits = jax.lax.dot_general(
        x, wt_ref[...], (((1,), (0,)), ((), ())),
        preferred_element_type=jnp.float32)

    m = jnp.max(logits, axis=-1, keepdims=True)
    ex = jnp.exp(logits - m)
    denom = jnp.sum(ex, axis=-1, keepdims=True)
    probs = ex / denom
    probs_ref[0] = probs

    maxp = 1.0 / denom
    lane = jax.lax.broadcasted_iota(jnp.int32, (T, E), 1).astype(jnp.float32)
    idx = jnp.min(jnp.where(probs == maxp, lane, float(E)),
                  axis=-1, keepdims=True)
    mask = (lane == idx).astype(jnp.float32)

    @pl.when(sblk == 0)
    def _():
        carry_ref[...] = jnp.zeros_like(carry_ref)

    run = carry_ref[...]
    for c in range(T // CHUNK):
        sl = slice(c * CHUNK, (c + 1) * CHUNK)
        csum_c = jax.lax.dot_general(
            tril_ref[...], mask[sl], (((1,), (0,)), ((), ())),
            preferred_element_type=jnp.float32) + run
        d_c = jnp.where(csum_c <= EXPERT_CAPACITY, mask[sl], 0.0)
        dispatch_ref[0, sl, :] = d_c
        combine_ref[0, sl, :] = d_c * maxp[sl]
        run = csum_c[CHUNK - 1:CHUNK, :]
    carry_ref[...] = run

    @pl.when((b == 0) & (sblk == 0))
    def _():
        acc_ref[0] = 0.0
        acc_ref[1] = 0.0

    lse = m + jnp.log(denom)
    acc_ref[0] += jnp.sum(probs * probs)
    acc_ref[1] += jnp.sum(lse * lse)
    aux_ref[...] = jnp.full((1, 1), acc_ref[0] * (E / total_tokens),
                            jnp.float32)
    z_ref[...] = jnp.full((1, 1), acc_ref[1] / total_tokens, jnp.float32)


@jax.jit
def kernel(hidden_states, W):
    B, S, D = hidden_states.shape
    E = W.shape[0]
    nsb = S // BLOCK_S
    wt = W.T
    r = jax.lax.broadcasted_iota(jnp.int32, (CHUNK, CHUNK), 0)
    c = jax.lax.broadcasted_iota(jnp.int32, (CHUNK, CHUNK), 1)
    tril = (r >= c).astype(jnp.float32)

    out_shapes = (
        jax.ShapeDtypeStruct((B, S, E), jnp.float32),
        jax.ShapeDtypeStruct((B, S, E), jnp.float32),
        jax.ShapeDtypeStruct((B, S, E), jnp.float32),
        jax.ShapeDtypeStruct((1, 1), jnp.float32),
        jax.ShapeDtypeStruct((1, 1), jnp.float32),
    )
    bse_spec = pl.BlockSpec((1, BLOCK_S, E), lambda b, s: (b, s, 0))
    scalar_spec = pl.BlockSpec((1, 1), lambda b, s: (0, 0))

    dispatch, combine, probs, aux, z = pl.pallas_call(
        functools.partial(_router_kernel, total_tokens=B * S),
        grid=(B, nsb),
        in_specs=[
            pl.BlockSpec((1, BLOCK_S, D), lambda b, s: (b, s, 0)),
            pl.BlockSpec((D, E), lambda b, s: (0, 0)),
            pl.BlockSpec((CHUNK, CHUNK), lambda b, s: (0, 0)),
        ],
        out_specs=(bse_spec, bse_spec, bse_spec, scalar_spec, scalar_spec),
        out_shape=out_shapes,
        scratch_shapes=[
            pltpu.VMEM((1, E), jnp.float32),
            pltpu.SMEM((2,), jnp.float32),
        ],
    )(hidden_states, wt, tril)

    return (dispatch, combine, probs, aux[0, 0], z[0, 0])
